# Initial kernel scaffold; baseline (speedup 1.0000x reference)
import jax
import jax.numpy as jnp
from jax.experimental import pallas as pl


def kernel(in_degrees, out_degrees, in_table, out_table):
    raise NotImplementedError("write your pallas kernel here")


# speedup vs baseline: 1.3986x; 1.3986x over previous
import jax
import jax.numpy as jnp
from jax import lax
from jax.experimental import pallas as pl
from jax.experimental.pallas import tpu as pltpu
from jax.experimental.pallas import tpu_sc as plsc

_N = 100000
_D = 128
_L = 16
_NC = 2
_NS = 16
_NW = _NC * _NS
_CHUNK = 128
_CHUNKS_PER_W = 25
_ROWS_PER_W = _CHUNK * _CHUNKS_PER_W
_N_PAD = _NW * _ROWS_PER_W


def _sc_body(in_deg, out_deg, in_tab, out_tab, out,
             idx_a, idx_b, rows_a, rows_b, sem_a, sem_b):
    wid = lax.axis_index("s") * _NC + lax.axis_index("c")
    base = wid * _ROWS_PER_W

    def chunk_body(k, carry):
        gbase = base + k * _CHUNK
        pltpu.sync_copy(in_deg.at[pl.ds(gbase, _CHUNK)], idx_a)
        pltpu.sync_copy(out_deg.at[pl.ds(gbase, _CHUNK)], idx_b)
        cp_a = pltpu.async_copy(in_tab.at[idx_a], rows_a, sem_a)
        cp_b = pltpu.async_copy(out_tab.at[idx_b], rows_b, sem_b)
        cp_a.wait()
        cp_b.wait()

        def row_body(r, carry2):
            for j in range(_D // _L):
                sl = pl.ds(j * _L, _L)
                rows_a[r, sl] = rows_a[r, sl] + rows_b[r, sl]
            return carry2

        lax.fori_loop(0, _CHUNK, row_body, 0, unroll=False)
        pltpu.sync_copy(rows_a, out.at[pl.ds(gbase, _CHUNK)])
        return carry

    lax.fori_loop(0, _CHUNKS_PER_W, chunk_body, 0, unroll=False)


def kernel(in_degrees, out_degrees, in_table, out_table):
    pad = _N_PAD - _N
    zeros = jnp.zeros((pad,), jnp.int32)
    in_idx = jnp.concatenate([in_degrees.astype(jnp.int32), zeros])
    out_idx = jnp.concatenate([out_degrees.astype(jnp.int32), zeros])

    mesh = plsc.VectorSubcoreMesh(core_axis_name="c", subcore_axis_name="s")
    out = pl.kernel(
        _sc_body,
        out_type=jax.ShapeDtypeStruct((_N_PAD, _D), jnp.float32),
        mesh=mesh,
        scratch_types=[
            pltpu.VMEM((_CHUNK,), jnp.int32),
            pltpu.VMEM((_CHUNK,), jnp.int32),
            pltpu.VMEM((_CHUNK, _D), jnp.float32),
            pltpu.VMEM((_CHUNK, _D), jnp.float32),
            pltpu.SemaphoreType.DMA,
            pltpu.SemaphoreType.DMA,
        ],
    )(in_idx, out_idx, in_table, out_table)
    return out[:_N]


# speedup vs baseline: 2.0510x; 1.4665x over previous
import jax
import jax.numpy as jnp
from jax import lax
from jax.experimental import pallas as pl
from jax.experimental.pallas import tpu as pltpu
from jax.experimental.pallas import tpu_sc as plsc

_N = 100000
_D = 128
_L = 16
_NC = 2
_NS = 16
_NW = _NC * _NS
_CHUNK = 128
_NCHUNK = 25
_ROWS_PER_W = _CHUNK * _NCHUNK
_N_PAD = _NW * _ROWS_PER_W
_TAIL = _N % _CHUNK


def _sc_body(in_deg, out_deg, in_tab, out_tab, out,
             idx_a, idx_b,
             ra0, ra1, ra2, rb0, rb1, rb2,
             sga0, sga1, sga2, sgb0, sgb1, sgb2, sw0, sw1, sw2):
    wid = lax.axis_index("s") * _NC + lax.axis_index("c")
    base = wid * _ROWS_PER_W
    ras = (ra0, ra1, ra2)
    rbs = (rb0, rb1, rb2)
    sgas = (sga0, sga1, sga2)
    sgbs = (sgb0, sgb1, sgb2)
    sws = (sw0, sw1, sw2)

    pltpu.sync_copy(in_deg.at[pl.ds(base, _ROWS_PER_W)], idx_a)
    pltpu.sync_copy(out_deg.at[pl.ds(base, _ROWS_PER_W)], idx_b)

    def issue_gather(c, s):
        sl = pl.ds(c * _CHUNK, _CHUNK)
        pltpu.async_copy(in_tab.at[idx_a.at[sl]], ras[s], sgas[s])
        pltpu.async_copy(out_tab.at[idx_b.at[sl]], rbs[s], sgbs[s])

    def wait_gather(s):
        sl = pl.ds(0, _CHUNK)
        pltpu.make_async_copy(in_tab.at[idx_a.at[sl]], ras[s], sgas[s]).wait()
        pltpu.make_async_copy(out_tab.at[idx_b.at[sl]], rbs[s], sgbs[s]).wait()

    def issue_write(c, s):
        gbase = base + c * _CHUNK

        @pl.when(gbase + _CHUNK <= _N)
        def _():
            pltpu.async_copy(ras[s], out.at[pl.ds(gbase, _CHUNK)], sws[s])

        @pl.when(jnp.logical_and(gbase < _N, gbase + _CHUNK > _N))
        def _():
            pltpu.async_copy(ras[s].at[pl.ds(0, _TAIL)],
                             out.at[pl.ds(gbase, _TAIL)], sws[s])

    def wait_write(c, s, extra_pred):
        gbase = base + c * _CHUNK
        p_full = jnp.logical_and(extra_pred, gbase + _CHUNK <= _N)
        p_part = jnp.logical_and(
            extra_pred, jnp.logical_and(gbase < _N, gbase + _CHUNK > _N))

        @pl.when(p_full)
        def _():
            pltpu.make_async_copy(ras[s], out.at[pl.ds(gbase, _CHUNK)],
                                  sws[s]).wait()

        @pl.when(p_part)
        def _():
            pltpu.make_async_copy(ras[s].at[pl.ds(0, _TAIL)],
                                  out.at[pl.ds(gbase, _TAIL)], sws[s]).wait()

    def accumulate(s):
        ra, rb = ras[s], rbs[s]

        def row_body(r, carry):
            for j in range(_D // _L):
                sl = pl.ds(j * _L, _L)
                plsc.addupdate(ra.at[r, sl], rb[r, sl])
            return carry

        lax.fori_loop(0, _CHUNK, row_body, 0)

    true_pred = jnp.bool_(True)
    issue_gather(0, 0)

    def loop_body(j, carry):
        for s in range(3):
            c = 3 * j + s
            sn = (s + 1) % 3
            wait_write(c - 2, sn, c >= 2)
            issue_gather(c + 1, sn)
            wait_gather(s)
            accumulate(s)
            issue_write(c, s)
        return carry

    lax.fori_loop(0, (_NCHUNK - 1) // 3, loop_body, 0)

    wait_gather(0)
    accumulate(0)
    issue_write(24, 0)
    wait_write(22, 1, true_pred)
    wait_write(23, 2, true_pred)
    wait_write(24, 0, true_pred)


def kernel(in_degrees, out_degrees, in_table, out_table):
    pad = _N_PAD - _N
    zeros = jnp.zeros((pad,), jnp.int32)
    in_idx = jnp.concatenate([in_degrees.astype(jnp.int32), zeros])
    out_idx = jnp.concatenate([out_degrees.astype(jnp.int32), zeros])

    mesh = plsc.VectorSubcoreMesh(core_axis_name="c", subcore_axis_name="s")
    return pl.kernel(
        _sc_body,
        out_type=jax.ShapeDtypeStruct((_N, _D), jnp.float32),
        mesh=mesh,
        scratch_types=[
            pltpu.VMEM((_ROWS_PER_W,), jnp.int32),
            pltpu.VMEM((_ROWS_PER_W,), jnp.int32),
            pltpu.VMEM((_CHUNK, _D), jnp.float32),
            pltpu.VMEM((_CHUNK, _D), jnp.float32),
            pltpu.VMEM((_CHUNK, _D), jnp.float32),
            pltpu.VMEM((_CHUNK, _D), jnp.float32),
            pltpu.VMEM((_CHUNK, _D), jnp.float32),
            pltpu.VMEM((_CHUNK, _D), jnp.float32),
            pltpu.SemaphoreType.DMA,
            pltpu.SemaphoreType.DMA,
            pltpu.SemaphoreType.DMA,
            pltpu.SemaphoreType.DMA,
            pltpu.SemaphoreType.DMA,
            pltpu.SemaphoreType.DMA,
            pltpu.SemaphoreType.DMA,
            pltpu.SemaphoreType.DMA,
            pltpu.SemaphoreType.DMA,
        ],
    )(in_idx, out_idx, in_table, out_table)
